# Initial kernel scaffold; baseline (speedup 1.0000x reference)
import jax
import jax.numpy as jnp
from jax.experimental import pallas as pl


def kernel(h, edge_index, edge_attr, mask, W1, b1, W2, b2, W3, b3, W4, b4):
    raise NotImplementedError("write your pallas kernel here")


# speedup vs baseline: 1.9827x; 1.9827x over previous
import functools

import jax
import jax.numpy as jnp
from jax import lax
from jax.experimental import pallas as pl
from jax.experimental.pallas import tpu as pltpu
from jax.experimental.pallas import tpu_sc as plsc

_N, _E, _D, _R = 10000, 160000, 256, 20
_H = _D // 2
_NS = 16
_CH = 80
_CPT = _E // (_NS * _CH)
_NG = 5
_CPG = _CPT // _NG
_NP = 10240
_RPT = _NP // _NS


def _silu(x):
    return x * jax.nn.sigmoid(x)


def _edge_mlp_body(ea, w1, b1, w2, b2, out):
    x = jnp.dot(ea[...], w1[...], preferred_element_type=jnp.float32) + b1[...]
    x = _silu(x)
    y = jnp.dot(x, w2[...], preferred_element_type=jnp.float32) + b2[...]
    y = _silu(y)
    out[0] = y[:, :_H]
    out[1] = y[:, _H:]


def _edge_mlp(edge_attr, W1, b1, W2, b2):
    Eb = 2000
    return pl.pallas_call(
        _edge_mlp_body,
        grid=(_E // Eb,),
        in_specs=[
            pl.BlockSpec((Eb, _R), lambda i: (i, 0)),
            pl.BlockSpec((_R, _D), lambda i: (0, 0)),
            pl.BlockSpec((1, _D), lambda i: (0, 0)),
            pl.BlockSpec((_D, _D), lambda i: (0, 0)),
            pl.BlockSpec((1, _D), lambda i: (0, 0)),
        ],
        out_specs=pl.BlockSpec((2, Eb, _H), lambda i: (0, i, 0)),
        out_shape=jax.ShapeDtypeStruct((2, _E, _H), jnp.float32),
    )(edge_attr, W1, b1.reshape(1, _D), W2, b2.reshape(1, _D))


def _sc_body(h2, ew, ridx_h, cidx_h, zrows, out, ridx, cidx, hbuf, ebuf, acc):
    c = lax.axis_index("c")
    s = lax.axis_index("s")
    pltpu.sync_copy(zrows, acc.at[pl.ds(s * _RPT, _RPT)])
    plsc.subcore_barrier()

    def group(g, carry):
        pltpu.sync_copy(ridx_h.at[c, s, g], ridx)
        pltpu.sync_copy(cidx_h.at[s, g], cidx)

        def chunk(j, carry1):
            e0 = c * _E + ((s * _NG + g) * _CPG + j) * _CH
            pltpu.sync_copy(ew.at[pl.ds(e0, _CH)], ebuf)
            pltpu.sync_copy(h2.at[ridx.at[j]], hbuf)

            def rowfn(r, carry2):
                for k in range(_H // 16):
                    sl = pl.ds(k * 16, 16)
                    hbuf[r, sl] = hbuf[r, sl] * ebuf[r, sl]
                return carry2

            lax.fori_loop(0, _CH, rowfn, 0)
            pltpu.sync_copy(hbuf, acc.at[cidx.at[j]], add=True)
            return carry1

        lax.fori_loop(0, _CPG, chunk, 0)
        return carry

    lax.fori_loop(0, _NG, group, 0)
    plsc.subcore_barrier()
    pltpu.sync_copy(acc.at[pl.ds(s * _RPT, _RPT)], out.at[c, s])


def _sc_scatter(h2, ew, ridx_h, cidx_h, zrows):
    mesh = plsc.VectorSubcoreMesh(core_axis_name="c", subcore_axis_name="s")
    run = functools.partial(
        pl.kernel,
        mesh=mesh,
        out_type=jax.ShapeDtypeStruct((2, _NS, _RPT, _H), jnp.float32),
        scratch_types=[
            pltpu.VMEM((_CPG, _CH), jnp.int32),
            pltpu.VMEM((_CPG, _CH), jnp.int32),
            pltpu.VMEM((_CH, _H), jnp.float32),
            pltpu.VMEM((_CH, _H), jnp.float32),
            pltpu.VMEM_SHARED((_NP, _H), jnp.float32),
        ],
    )(_sc_body)
    return run(h2, ew, ridx_h, cidx_h, zrows)


def _out_mlp_body(h, n0, n1, w3h, w3n0, w3n1, b3, w4, b4, o):
    t = (jnp.dot(h[...], w3h[...], preferred_element_type=jnp.float32)
         + jnp.dot(n0[...], w3n0[...], preferred_element_type=jnp.float32)
         + jnp.dot(n1[...], w3n1[...], preferred_element_type=jnp.float32)
         + b3[...])
    t = _silu(t)
    o[...] = h[...] + jnp.dot(t, w4[...], preferred_element_type=jnp.float32) + b4[...]


def _out_mlp(h, n0, n1, W3, b3, W4, b4):
    Nb = 2000
    full = lambda i: (0, 0)
    return pl.pallas_call(
        _out_mlp_body,
        grid=(_N // Nb,),
        in_specs=[
            pl.BlockSpec((Nb, _D), lambda i: (i, 0)),
            pl.BlockSpec((Nb, _H), lambda i: (i, 0)),
            pl.BlockSpec((Nb, _H), lambda i: (i, 0)),
            pl.BlockSpec((_D, _D), full),
            pl.BlockSpec((_H, _D), full),
            pl.BlockSpec((_H, _D), full),
            pl.BlockSpec((1, _D), full),
            pl.BlockSpec((_D, _D), full),
            pl.BlockSpec((1, _D), full),
        ],
        out_specs=pl.BlockSpec((Nb, _D), lambda i: (i, 0)),
        out_shape=jax.ShapeDtypeStruct((_N, _D), jnp.float32),
    )(h, n0, n1, W3[:_D], W3[_D:_D + _H], W3[_D + _H:], b3.reshape(1, _D),
      W4, b4.reshape(1, _D))


def kernel(h, edge_index, edge_attr, mask, W1, b1, W2, b2, W3, b3, W4, b4):
    row = edge_index[0]
    col = edge_index[1]
    ew2 = _edge_mlp(edge_attr, W1, b1, W2, b2).reshape(2 * _E, _H)
    h2 = jnp.concatenate([h[:, :_H], h[:, _H:]], axis=0)
    ridx_h = jnp.stack([row, row + _N]).reshape(2, _NS, _NG, _CPG, _CH)
    cidx_h = col.reshape(_NS, _NG, _CPG, _CH)
    zrows = jnp.zeros((_RPT, _H), jnp.float32)
    hn4 = _sc_scatter(h2, ew2, ridx_h, cidx_h, zrows)
    hn = hn4.reshape(2, _NP, _H)[:, :_N]
    return _out_mlp(h, hn[0], hn[1], W3, b3, W4, b4)


# speedup vs baseline: 2.7860x; 1.4051x over previous
import functools

import jax
import jax.numpy as jnp
from jax import lax
from jax.experimental import pallas as pl
from jax.experimental.pallas import tpu as pltpu
from jax.experimental.pallas import tpu_sc as plsc

_N, _E, _D, _R = 10000, 160000, 256, 20
_H = _D // 2
_NS = 16
_CH = 40
_CPT = _E // (_NS * _CH)
_NG = 5
_CPG = _CPT // _NG
_NP = 10240
_RPT = _NP // _NS


def _silu(x):
    return x * jax.nn.sigmoid(x)


def _edge_mlp_body(ea, w1, b1, w2, b2, out):
    x = jnp.dot(ea[...], w1[...], preferred_element_type=jnp.float32) + b1[...]
    x = _silu(x)
    y = jnp.dot(x, w2[...], preferred_element_type=jnp.float32) + b2[...]
    y = _silu(y)
    out[0] = y[:, :_H]
    out[1] = y[:, _H:]


def _edge_mlp(edge_attr, W1, b1, W2, b2):
    Eb = 2000
    return pl.pallas_call(
        _edge_mlp_body,
        grid=(_E // Eb,),
        in_specs=[
            pl.BlockSpec((Eb, _R), lambda i: (i, 0)),
            pl.BlockSpec((_R, _D), lambda i: (0, 0)),
            pl.BlockSpec((1, _D), lambda i: (0, 0)),
            pl.BlockSpec((_D, _D), lambda i: (0, 0)),
            pl.BlockSpec((1, _D), lambda i: (0, 0)),
        ],
        out_specs=pl.BlockSpec((2, Eb, _H), lambda i: (0, i, 0)),
        out_shape=jax.ShapeDtypeStruct((2, _E, _H), jnp.float32),
    )(edge_attr, W1, b1.reshape(1, _D), W2, b2.reshape(1, _D))


def _sc_body(h2, ew, ridx_h, cidx_h, zrows, out, ridx, cidx, hbuf, ebuf, mbuf,
             acc, hsem, esem):
    c = lax.axis_index("c")
    s = lax.axis_index("s")
    pltpu.sync_copy(zrows, acc.at[pl.ds(s * _RPT, _RPT)])
    plsc.subcore_barrier()

    def _e0(g, j):
        return c * _E + (s * _CPT + g * _CPG + j) * _CH

    def _start(g, j, slot):
        pltpu.async_copy(ew.at[pl.ds(_e0(g, j), _CH)], ebuf.at[slot], esem)
        pltpu.async_copy(h2.at[ridx.at[j]], hbuf.at[slot], hsem)

    def _finish(g, j, slot):
        pltpu.make_async_copy(ew.at[pl.ds(_e0(g, j), _CH)], ebuf.at[slot],
                              esem).wait()
        pltpu.make_async_copy(h2.at[ridx.at[j]], hbuf.at[slot], hsem).wait()

    def _consume(g, j, slot):
        _finish(g, j, slot)

        def rowfn(r, carry2):
            for k in range(_H // 16):
                sl = pl.ds(k * 16, 16)
                mbuf[r, sl] = hbuf[slot, r, sl] * ebuf[slot, r, sl]
            return carry2

        lax.fori_loop(0, _CH, rowfn, 0)
        pltpu.sync_copy(mbuf, acc.at[cidx.at[j]], add=True)

    def group(g, carry):
        pltpu.sync_copy(ridx_h.at[c, s, g], ridx)
        pltpu.sync_copy(cidx_h.at[s, g], cidx)
        _start(g, 0, 0)

        def pair(p, carry1):
            j0 = 2 * p
            _start(g, j0 + 1, 1)
            _consume(g, j0, 0)

            @pl.when(j0 + 2 < _CPG)
            def _():
                _start(g, j0 + 2, 0)

            _consume(g, j0 + 1, 1)
            return carry1

        lax.fori_loop(0, _CPG // 2, pair, 0)
        return carry

    lax.fori_loop(0, _NG, group, 0)
    plsc.subcore_barrier()
    pltpu.sync_copy(acc.at[pl.ds(s * _RPT, _RPT)], out.at[c, s])


def _sc_scatter(h2, ew, ridx_h, cidx_h, zrows):
    mesh = plsc.VectorSubcoreMesh(core_axis_name="c", subcore_axis_name="s")
    run = functools.partial(
        pl.kernel,
        mesh=mesh,
        out_type=jax.ShapeDtypeStruct((2, _NS, _RPT, _H), jnp.float32),
        scratch_types=[
            pltpu.VMEM((_CPG, _CH), jnp.int32),
            pltpu.VMEM((_CPG, _CH), jnp.int32),
            pltpu.VMEM((2, _CH, _H), jnp.float32),
            pltpu.VMEM((2, _CH, _H), jnp.float32),
            pltpu.VMEM((_CH, _H), jnp.float32),
            pltpu.VMEM_SHARED((_NP, _H), jnp.float32),
            pltpu.SemaphoreType.DMA,
            pltpu.SemaphoreType.DMA,
        ],
    )(_sc_body)
    return run(h2, ew, ridx_h, cidx_h, zrows)


def _out_mlp_body(h, n0, n1, w3h, w3n0, w3n1, b3, w4, b4, o):
    t = (jnp.dot(h[...], w3h[...], preferred_element_type=jnp.float32)
         + jnp.dot(n0[...], w3n0[...], preferred_element_type=jnp.float32)
         + jnp.dot(n1[...], w3n1[...], preferred_element_type=jnp.float32)
         + b3[...])
    t = _silu(t)
    o[...] = h[...] + jnp.dot(t, w4[...], preferred_element_type=jnp.float32) + b4[...]


def _out_mlp(h, n0, n1, W3, b3, W4, b4):
    Nb = 2000
    full = lambda i: (0, 0)
    return pl.pallas_call(
        _out_mlp_body,
        grid=(_N // Nb,),
        in_specs=[
            pl.BlockSpec((Nb, _D), lambda i: (i, 0)),
            pl.BlockSpec((Nb, _H), lambda i: (i, 0)),
            pl.BlockSpec((Nb, _H), lambda i: (i, 0)),
            pl.BlockSpec((_D, _D), full),
            pl.BlockSpec((_H, _D), full),
            pl.BlockSpec((_H, _D), full),
            pl.BlockSpec((1, _D), full),
            pl.BlockSpec((_D, _D), full),
            pl.BlockSpec((1, _D), full),
        ],
        out_specs=pl.BlockSpec((Nb, _D), lambda i: (i, 0)),
        out_shape=jax.ShapeDtypeStruct((_N, _D), jnp.float32),
    )(h, n0, n1, W3[:_D], W3[_D:_D + _H], W3[_D + _H:], b3.reshape(1, _D),
      W4, b4.reshape(1, _D))


def kernel(h, edge_index, edge_attr, mask, W1, b1, W2, b2, W3, b3, W4, b4):
    row = edge_index[0]
    col = edge_index[1]
    ew2 = _edge_mlp(edge_attr, W1, b1, W2, b2).reshape(2 * _E, _H)
    h2 = jnp.concatenate([h[:, :_H], h[:, _H:]], axis=0)
    ridx_h = jnp.stack([row, row + _N]).reshape(2, _NS, _NG, _CPG, _CH)
    cidx_h = col.reshape(_NS, _NG, _CPG, _CH)
    zrows = jnp.zeros((_RPT, _H), jnp.float32)
    hn4 = _sc_scatter(h2, ew2, ridx_h, cidx_h, zrows)
    hn = hn4.reshape(2, _NP, _H)[:, :_N]
    return _out_mlp(h, hn[0], hn[1], W3, b3, W4, b4)


# speedup vs baseline: 2.9327x; 1.0527x over previous
import functools

import jax
import jax.numpy as jnp
from jax import lax
from jax.experimental import pallas as pl
from jax.experimental.pallas import tpu as pltpu
from jax.experimental.pallas import tpu_sc as plsc

_N, _E, _D, _R = 10000, 160000, 256, 20
_H = _D // 2
_NS = 16
_CH = 40
_CPT = _E // (_NS * _CH)
_NG = 5
_CPG = _CPT // _NG
_NP = 10240
_RPT = _NP // _NS


def _silu(x):
    return 0.5 * x * (1.0 + jnp.tanh(0.5 * x))


def _edge_mlp_body(ea, w1, b1, w2, b2, out):
    x = jnp.dot(ea[...], w1[...], preferred_element_type=jnp.float32) + b1[...]
    x = _silu(x)
    y = jnp.dot(x.astype(jnp.bfloat16), w2[...],
                preferred_element_type=jnp.float32) + b2[...]
    y = _silu(y)
    out[0] = y[:, :_H]
    out[1] = y[:, _H:]


def _edge_mlp(edge_attr, W1, b1, W2, b2):
    Eb = 2000
    return pl.pallas_call(
        _edge_mlp_body,
        grid=(_E // Eb,),
        in_specs=[
            pl.BlockSpec((Eb, _R), lambda i: (i, 0)),
            pl.BlockSpec((_R, _D), lambda i: (0, 0)),
            pl.BlockSpec((1, _D), lambda i: (0, 0)),
            pl.BlockSpec((_D, _D), lambda i: (0, 0)),
            pl.BlockSpec((1, _D), lambda i: (0, 0)),
        ],
        out_specs=pl.BlockSpec((2, Eb, _H), lambda i: (0, i, 0)),
        out_shape=jax.ShapeDtypeStruct((2, _E, _H), jnp.float32),
    )(edge_attr.astype(jnp.bfloat16), W1.astype(jnp.bfloat16),
      b1.reshape(1, _D), W2.astype(jnp.bfloat16), b2.reshape(1, _D))


def _sc_body(h2, ew, ridx_h, cidx_h, zrows, out, ridx, cidx, hbuf, ebuf, mbuf,
             acc, hsem, esem):
    c = lax.axis_index("c")
    s = lax.axis_index("s")
    pltpu.sync_copy(zrows, acc.at[pl.ds(s * _RPT, _RPT)])
    plsc.subcore_barrier()

    def _e0(g, j):
        return c * _E + (s * _CPT + g * _CPG + j) * _CH

    def _start(g, j, slot):
        pltpu.async_copy(ew.at[pl.ds(_e0(g, j), _CH)], ebuf.at[slot], esem)
        pltpu.async_copy(h2.at[ridx.at[j]], hbuf.at[slot], hsem)

    def _finish(g, j, slot):
        pltpu.make_async_copy(ew.at[pl.ds(_e0(g, j), _CH)], ebuf.at[slot],
                              esem).wait()
        pltpu.make_async_copy(h2.at[ridx.at[j]], hbuf.at[slot], hsem).wait()

    def _consume(g, j, slot):
        _finish(g, j, slot)

        def rowfn(r, carry2):
            for k in range(_H // 16):
                sl = pl.ds(k * 16, 16)
                mbuf[r, sl] = hbuf[slot, r, sl] * ebuf[slot, r, sl]
            return carry2

        lax.fori_loop(0, _CH, rowfn, 0)
        pltpu.sync_copy(mbuf, acc.at[cidx.at[j]], add=True)

    def group(g, carry):
        pltpu.sync_copy(ridx_h.at[c, s, g], ridx)
        pltpu.sync_copy(cidx_h.at[s, g], cidx)
        _start(g, 0, 0)

        def pair(p, carry1):
            j0 = 2 * p
            _start(g, j0 + 1, 1)
            _consume(g, j0, 0)

            @pl.when(j0 + 2 < _CPG)
            def _():
                _start(g, j0 + 2, 0)

            _consume(g, j0 + 1, 1)
            return carry1

        lax.fori_loop(0, _CPG // 2, pair, 0)
        return carry

    lax.fori_loop(0, _NG, group, 0)
    plsc.subcore_barrier()
    pltpu.sync_copy(acc.at[pl.ds(s * _RPT, _RPT)], out.at[c, s])


def _sc_scatter(h2, ew, ridx_h, cidx_h, zrows):
    mesh = plsc.VectorSubcoreMesh(core_axis_name="c", subcore_axis_name="s")
    run = functools.partial(
        pl.kernel,
        mesh=mesh,
        out_type=jax.ShapeDtypeStruct((2, _NS, _RPT, _H), jnp.float32),
        scratch_types=[
            pltpu.VMEM((_CPG, _CH), jnp.int32),
            pltpu.VMEM((_CPG, _CH), jnp.int32),
            pltpu.VMEM((2, _CH, _H), jnp.float32),
            pltpu.VMEM((2, _CH, _H), jnp.float32),
            pltpu.VMEM((_CH, _H), jnp.float32),
            pltpu.VMEM_SHARED((_NP, _H), jnp.float32),
            pltpu.SemaphoreType.DMA,
            pltpu.SemaphoreType.DMA,
        ],
    )(_sc_body)
    return run(h2, ew, ridx_h, cidx_h, zrows)


def _out_mlp_body(h, n0, n1, w3h, w3n0, w3n1, b3, w4, b4, o):
    t = (jnp.dot(h[...], w3h[...], preferred_element_type=jnp.float32)
         + jnp.dot(n0[...], w3n0[...], preferred_element_type=jnp.float32)
         + jnp.dot(n1[...], w3n1[...], preferred_element_type=jnp.float32)
         + b3[...])
    t = _silu(t)
    o[...] = h[...] + jnp.dot(t, w4[...], preferred_element_type=jnp.float32) + b4[...]


def _out_mlp(h, n0, n1, W3, b3, W4, b4):
    Nb = 2000
    full = lambda i: (0, 0)
    return pl.pallas_call(
        _out_mlp_body,
        grid=(_N // Nb,),
        in_specs=[
            pl.BlockSpec((Nb, _D), lambda i: (i, 0)),
            pl.BlockSpec((Nb, _H), lambda i: (i, 0)),
            pl.BlockSpec((Nb, _H), lambda i: (i, 0)),
            pl.BlockSpec((_D, _D), full),
            pl.BlockSpec((_H, _D), full),
            pl.BlockSpec((_H, _D), full),
            pl.BlockSpec((1, _D), full),
            pl.BlockSpec((_D, _D), full),
            pl.BlockSpec((1, _D), full),
        ],
        out_specs=pl.BlockSpec((Nb, _D), lambda i: (i, 0)),
        out_shape=jax.ShapeDtypeStruct((_N, _D), jnp.float32),
    )(h, n0, n1, W3[:_D], W3[_D:_D + _H], W3[_D + _H:], b3.reshape(1, _D),
      W4, b4.reshape(1, _D))


def kernel(h, edge_index, edge_attr, mask, W1, b1, W2, b2, W3, b3, W4, b4):
    row = edge_index[0]
    col = edge_index[1]
    ew2 = _edge_mlp(edge_attr, W1, b1, W2, b2).reshape(2 * _E, _H)
    h2 = jnp.concatenate([h[:, :_H], h[:, _H:]], axis=0)
    ridx_h = jnp.stack([row, row + _N]).reshape(2, _NS, _NG, _CPG, _CH)
    cidx_h = col.reshape(_NS, _NG, _CPG, _CH)
    zrows = jnp.zeros((_RPT, _H), jnp.float32)
    hn4 = _sc_scatter(h2, ew2, ridx_h, cidx_h, zrows)
    hn = hn4.reshape(2, _NP, _H)[:, :_N]
    return _out_mlp(h, hn[0], hn[1], W3, b3, W4, b4)
